# baseline (speedup 1.0000x reference)
import jax
import jax.numpy as jnp
from jax.experimental import pallas as pl
from jax.experimental.pallas import tpu as pltpu

_BATCH = 16384
_DIM = 256
_SIZES = (512, 512, 1024, 2048, 2048, 2048, 2048, 2048, 2048, 1024, 512, 512)
_CHUNKS = []
_off = 0
for _sz in _SIZES:
    _CHUNKS.append((_off, _sz))
    _off += _sz
assert _off == _BATCH
_NCH = len(_CHUNKS)


def _body(gu_hbm, gi_hbm, xui_hbm, guo_hbm, gio_hbm,
          gu_v, gi_v, xui_v, insem, outsem, xsem):
    ins = []
    for c, (off, sz) in enumerate(_CHUNKS):
        a = pltpu.make_async_copy(
            gu_hbm.at[pl.ds(off, sz)], gu_v.at[pl.ds(off, sz)], insem.at[c])
        a.start()
        b = pltpu.make_async_copy(
            gi_hbm.at[pl.ds(off, sz)], gi_v.at[pl.ds(off, sz)], insem.at[c])
        b.start()
        ins.append((a, b))

    outs = []
    for c, (off, sz) in enumerate(_CHUNKS):
        a, b = ins[c]
        a.wait()
        b.wait()
        oa = pltpu.make_async_copy(
            gu_v.at[pl.ds(off, sz)], guo_hbm.at[pl.ds(off, sz)], outsem.at[c])
        oa.start()
        ob = pltpu.make_async_copy(
            gi_v.at[pl.ds(off, sz)], gio_hbm.at[pl.ds(off, sz)], outsem.at[c])
        ob.start()
        xui_v[pl.ds(off, sz)] = jnp.sum(
            gu_v[pl.ds(off, sz), :] * gi_v[pl.ds(off, sz), :], axis=1)
        ox = pltpu.make_async_copy(
            xui_v.at[pl.ds(off, sz)], xui_hbm.at[pl.ds(off, sz)], xsem.at[c])
        ox.start()
        outs += [oa, ob, ox]

    for h in outs:
        h.wait()


def kernel(gu, gi):
    xui, gu_out, gi_out = pl.pallas_call(
        _body,
        in_specs=[
            pl.BlockSpec(memory_space=pl.ANY),
            pl.BlockSpec(memory_space=pl.ANY),
        ],
        out_specs=[
            pl.BlockSpec(memory_space=pl.ANY),
            pl.BlockSpec(memory_space=pl.ANY),
            pl.BlockSpec(memory_space=pl.ANY),
        ],
        out_shape=[
            jax.ShapeDtypeStruct((_BATCH,), jnp.float32),
            jax.ShapeDtypeStruct((_BATCH, _DIM), jnp.float32),
            jax.ShapeDtypeStruct((_BATCH, _DIM), jnp.float32),
        ],
        scratch_shapes=[
            pltpu.VMEM((_BATCH, _DIM), jnp.float32),
            pltpu.VMEM((_BATCH, _DIM), jnp.float32),
            pltpu.VMEM((_BATCH,), jnp.float32),
            pltpu.SemaphoreType.DMA((_NCH,)),
            pltpu.SemaphoreType.DMA((_NCH,)),
            pltpu.SemaphoreType.DMA((_NCH,)),
        ],
    )(gu, gi)
    return (xui, gu_out, gi_out)


# speedup vs baseline: 1.0251x; 1.0251x over previous
import jax
import jax.numpy as jnp
from jax.experimental import pallas as pl
from jax.experimental.pallas import tpu as pltpu

_BATCH = 16384
_DIM = 256
_SIZES = (512, 2048, 4096, 4096, 4096, 1024, 512)
_CHUNKS = []
_off = 0
for _sz in _SIZES:
    _CHUNKS.append((_off, _sz))
    _off += _sz
assert _off == _BATCH
_NCH = len(_CHUNKS)


def _body(gu_hbm, gi_hbm, xui_hbm, guo_hbm, gio_hbm,
          gu_v, gi_v, xui_v, insem, outsem, xsem):
    ins = []
    for c, (off, sz) in enumerate(_CHUNKS):
        a = pltpu.make_async_copy(
            gu_hbm.at[pl.ds(off, sz)], gu_v.at[pl.ds(off, sz)], insem.at[c])
        a.start()
        b = pltpu.make_async_copy(
            gi_hbm.at[pl.ds(off, sz)], gi_v.at[pl.ds(off, sz)], insem.at[c])
        b.start()
        ins.append((a, b))

    outs = []
    for c, (off, sz) in enumerate(_CHUNKS):
        a, b = ins[c]
        a.wait()
        b.wait()
        oa = pltpu.make_async_copy(
            gu_v.at[pl.ds(off, sz)], guo_hbm.at[pl.ds(off, sz)], outsem.at[c])
        oa.start()
        ob = pltpu.make_async_copy(
            gi_v.at[pl.ds(off, sz)], gio_hbm.at[pl.ds(off, sz)], outsem.at[c])
        ob.start()
        xui_v[pl.ds(off, sz)] = jnp.sum(
            gu_v[pl.ds(off, sz), :] * gi_v[pl.ds(off, sz), :], axis=1)
        ox = pltpu.make_async_copy(
            xui_v.at[pl.ds(off, sz)], xui_hbm.at[pl.ds(off, sz)], xsem.at[c])
        ox.start()
        outs += [oa, ob, ox]

    for h in outs:
        h.wait()


def kernel(gu, gi):
    xui, gu_out, gi_out = pl.pallas_call(
        _body,
        in_specs=[
            pl.BlockSpec(memory_space=pl.ANY),
            pl.BlockSpec(memory_space=pl.ANY),
        ],
        out_specs=[
            pl.BlockSpec(memory_space=pl.ANY),
            pl.BlockSpec(memory_space=pl.ANY),
            pl.BlockSpec(memory_space=pl.ANY),
        ],
        out_shape=[
            jax.ShapeDtypeStruct((_BATCH,), jnp.float32),
            jax.ShapeDtypeStruct((_BATCH, _DIM), jnp.float32),
            jax.ShapeDtypeStruct((_BATCH, _DIM), jnp.float32),
        ],
        scratch_shapes=[
            pltpu.VMEM((_BATCH, _DIM), jnp.float32),
            pltpu.VMEM((_BATCH, _DIM), jnp.float32),
            pltpu.VMEM((_BATCH,), jnp.float32),
            pltpu.SemaphoreType.DMA((_NCH,)),
            pltpu.SemaphoreType.DMA((_NCH,)),
            pltpu.SemaphoreType.DMA((_NCH,)),
        ],
    )(gu, gi)
    return (xui, gu_out, gi_out)
